# baseline (speedup 1.0000x reference)
import functools
import numpy as np
import jax
import jax.numpy as jnp
from jax import lax
from jax.experimental import pallas as pl
from jax.experimental.pallas import tpu as pltpu
from jax.experimental.pallas import tpu_sc as plsc

_VOCAB = 100000
_D = 768
_MAX_SEQ = 2048
_SCALE = float(np.sqrt(np.float32(_D)))

_NC = 2
_NS = 16
_NW = _NC * _NS
_LANES = 16

_CHUNK = 32


def _sinus_pe(max_len, d_model):
    pos = np.arange(max_len, dtype=np.float32)[:, None]
    div = np.exp(np.arange(0, d_model, 2, dtype=np.float32) * (-np.log(10000.0) / d_model))
    pe = np.zeros((max_len, d_model), dtype=np.float32)
    pe[:, 0::2] = np.sin(pos * div)
    pe[:, 1::2] = np.cos(pos * div)
    return pe


_PE = _sinus_pe(_MAX_SEQ, _D)


def _make_emb_kernel(batch, seq_len):
    n_rows = batch * seq_len
    assert n_rows % _NW == 0
    rows_per_w = n_rows // _NW
    assert rows_per_w % _CHUNK == 0
    n_chunks = rows_per_w // _CHUNK
    assert seq_len % rows_per_w == 0

    mesh = plsc.VectorSubcoreMesh(
        core_axis_name="c", subcore_axis_name="s",
        num_cores=_NC, num_subcores=_NS)

    @functools.partial(
        pl.kernel,
        out_type=jax.ShapeDtypeStruct((n_rows, _D), jnp.float32),
        mesh=mesh,
        scratch_types=[
            pltpu.VMEM((rows_per_w,), jnp.int32),
            pltpu.VMEM((_CHUNK, _D), jnp.float32),
            pltpu.VMEM((_CHUNK, _D), jnp.float32),
            pltpu.SemaphoreType.DMA,
        ],
    )
    def emb(x_hbm, pe_hbm, table_hbm, out_hbm, idx_v, rows_v, pe_v, sem):
        wid = lax.axis_index("s") * _NC + lax.axis_index("c")
        base = wid * rows_per_w
        s_base = lax.rem(base, seq_len)
        pltpu.sync_copy(x_hbm.at[pl.ds(base, rows_per_w)], idx_v)
        for c in range(n_chunks):
            r0 = c * _CHUNK
            gather = pltpu.async_copy(
                table_hbm.at[idx_v.at[pl.ds(r0, _CHUNK)]], rows_v, sem)
            pltpu.sync_copy(pe_hbm.at[pl.ds(s_base + r0, _CHUNK)], pe_v)
            gather.wait()

            def row_body(r, _):
                def col_body(j, _):
                    sl = pl.ds(j * _LANES, _LANES)
                    rows_v[r, sl] = rows_v[r, sl] * _SCALE + pe_v[r, sl]
                    return 0
                return lax.fori_loop(0, _D // _LANES, col_body, 0)

            lax.fori_loop(0, _CHUNK, row_body, 0)
            pltpu.sync_copy(rows_v, out_hbm.at[pl.ds(base + r0, _CHUNK)])

    return emb


@jax.jit
def kernel(x, table):
    batch, seq_len = x.shape
    x_flat = x.reshape(-1).astype(jnp.int32)
    pe = jnp.asarray(_PE[:seq_len])
    out = _make_emb_kernel(batch, seq_len)(x_flat, pe, table)
    return out.reshape(batch, seq_len, _D)


# speedup vs baseline: 2.1767x; 2.1767x over previous
import functools
import numpy as np
import jax
import jax.numpy as jnp
from jax import lax
from jax.experimental import pallas as pl
from jax.experimental.pallas import tpu as pltpu
from jax.experimental.pallas import tpu_sc as plsc

_VOCAB = 100000
_D = 768
_MAX_SEQ = 2048
_SCALE = float(np.sqrt(np.float32(_D)))

_NC = 2
_NS = 16
_NW = _NC * _NS
_LANES = 16

_CHUNK = 32


def _sinus_pe(max_len, d_model):
    pos = np.arange(max_len, dtype=np.float32)[:, None]
    div = np.exp(np.arange(0, d_model, 2, dtype=np.float32) * (-np.log(10000.0) / d_model))
    pe = np.zeros((max_len, d_model), dtype=np.float32)
    pe[:, 0::2] = np.sin(pos * div)
    pe[:, 1::2] = np.cos(pos * div)
    return pe


_PE = _sinus_pe(_MAX_SEQ, _D)


def _make_emb_kernel(batch, seq_len):
    n_rows = batch * seq_len
    assert n_rows % _NW == 0
    rows_per_w = n_rows // _NW
    assert rows_per_w % _CHUNK == 0
    n_chunks = rows_per_w // _CHUNK
    assert seq_len % rows_per_w == 0

    mesh = plsc.VectorSubcoreMesh(
        core_axis_name="c", subcore_axis_name="s",
        num_cores=_NC, num_subcores=_NS)

    @functools.partial(
        pl.kernel,
        out_type=jax.ShapeDtypeStruct((n_rows, _D), jnp.float32),
        mesh=mesh,
        scratch_types=[
            pltpu.VMEM((rows_per_w,), jnp.int32),
            [pltpu.VMEM((_CHUNK, _D), jnp.float32) for _ in range(2)],
            [pltpu.VMEM((_CHUNK, _D), jnp.float32) for _ in range(2)],
            [pltpu.SemaphoreType.DMA for _ in range(2)],
            [pltpu.SemaphoreType.DMA for _ in range(2)],
            [pltpu.SemaphoreType.DMA for _ in range(2)],
        ],
    )
    def emb(x_hbm, pe_hbm, table_hbm, out_hbm,
            idx_v, rows_v, pe_v, gsem, psem, osem):
        wid = lax.axis_index("s") * _NC + lax.axis_index("c")
        base = wid * rows_per_w
        s_base = lax.rem(base, seq_len)
        pltpu.sync_copy(x_hbm.at[pl.ds(base, rows_per_w)], idx_v)

        def start_chunk(c):
            b = c % 2
            r0 = c * _CHUNK
            pltpu.async_copy(
                table_hbm.at[idx_v.at[pl.ds(r0, _CHUNK)]], rows_v[b], gsem[b])
            pltpu.async_copy(
                pe_hbm.at[pl.ds(s_base + r0, _CHUNK)], pe_v[b], psem[b])

        def wait_store(c):
            b = c % 2
            pltpu.make_async_copy(
                rows_v[b], out_hbm.at[pl.ds(base + c * _CHUNK, _CHUNK)],
                osem[b]).wait()

        start_chunk(0)
        start_chunk(1)
        for c in range(n_chunks):
            b = c % 2
            r0 = c * _CHUNK
            pltpu.make_async_copy(
                table_hbm.at[idx_v.at[pl.ds(r0, _CHUNK)]], rows_v[b],
                gsem[b]).wait()
            pltpu.make_async_copy(
                pe_hbm.at[pl.ds(s_base + r0, _CHUNK)], pe_v[b],
                psem[b]).wait()

            def row_body(r, _, b=b):
                for j in range(_D // _LANES):
                    sl = pl.ds(j * _LANES, _LANES)
                    rows_v[b][r, sl] = rows_v[b][r, sl] * _SCALE + pe_v[b][r, sl]
                return 0

            lax.fori_loop(0, _CHUNK, row_body, 0)
            pltpu.async_copy(
                rows_v[b], out_hbm.at[pl.ds(base + r0, _CHUNK)], osem[b])
            if c + 2 < n_chunks:
                wait_store(c)
                start_chunk(c + 2)
        wait_store(n_chunks - 2)
        wait_store(n_chunks - 1)

    return emb


@jax.jit
def kernel(x, table):
    batch, seq_len = x.shape
    x_flat = x.reshape(-1).astype(jnp.int32)
    pe = jnp.asarray(_PE[:seq_len])
    out = _make_emb_kernel(batch, seq_len)(x_flat, pe, table)
    return out.reshape(batch, seq_len, _D)
